# baseline (speedup 1.0000x reference)
import functools

import jax
import jax.numpy as jnp
from jax import lax
from jax.experimental import pallas as pl
from jax.experimental.pallas import tpu as pltpu, tpu_sc as plsc

NC = 2
NS = 16
NW = NC * NS

TOKENS_SHAPE = (16384, 20)
B = TOKENS_SHAPE[0] * TOKENS_SHAPE[1]
D = 64

CHUNK = 128
B_PER_W = B // NW
NCH = B_PER_W // CHUNK
NBUF = 4
N_GROUPS = NCH // NBUF - 1


def _emb_body(table_hbm, tok_hbm, out_hbm, idx_v, bufs, s0, s1, s2, s3):
    sems = (s0, s1, s2, s3)
    wid = lax.axis_index("s") * NC + lax.axis_index("c")
    base = wid * B_PER_W

    pltpu.sync_copy(tok_hbm.at[pl.ds(wid * NCH, NCH)], idx_v)

    def gather(j, b):
        return pltpu.make_async_copy(
            table_hbm.at[idx_v.at[j]], bufs.at[b], sems[b])

    for b in range(NBUF):
        gather(b, b).start()

    @pl.loop(0, N_GROUPS)
    def _(g):
        j0 = g * NBUF
        for b in range(NBUF):
            j = j0 + b
            gather(j, b).wait()
            pltpu.sync_copy(
                bufs.at[b], out_hbm.at[pl.ds(base + j * CHUNK, CHUNK)])
            gather(j + NBUF, b).start()

    for b in range(NBUF):
        j = N_GROUPS * NBUF + b
        gather(j, b).wait()
        pltpu.sync_copy(
            bufs.at[b], out_hbm.at[pl.ds(base + j * CHUNK, CHUNK)])


@jax.jit
def _emb_lookup(tok2d, emb_weight):
    mesh = plsc.VectorSubcoreMesh(core_axis_name="c", subcore_axis_name="s")
    run = pl.kernel(
        _emb_body,
        out_type=jax.ShapeDtypeStruct((B, D), jnp.float32),
        mesh=mesh,
        scratch_types=[
            pltpu.VMEM((NCH, CHUNK), jnp.int32),
            pltpu.VMEM((NBUF, CHUNK, D), jnp.float32),
            pltpu.SemaphoreType.DMA,
            pltpu.SemaphoreType.DMA,
            pltpu.SemaphoreType.DMA,
            pltpu.SemaphoreType.DMA,
        ],
        compiler_params=pltpu.CompilerParams(use_tc_tiling_on_sc=False),
    )
    return run(emb_weight, tok2d)


def kernel(tokens, emb_weight):
    tok2d = tokens.reshape(NW * NCH, CHUNK).astype(jnp.int32)
    out = _emb_lookup(tok2d, emb_weight)
    return out.reshape(*TOKENS_SHAPE, D)
